# baseline (speedup 1.0000x reference)
import jax
import jax.numpy as jnp
from jax import lax
from jax.experimental import pallas as pl
from jax.experimental.pallas import tpu as pltpu
from jax.experimental.pallas import tpu_sc as plsc

H = 768
E = 8
TOP_K = 2
HFF = 3072
LB_WEIGHT = 0.01
LN_EPS = 1e-5

T = 2048
TK = T * TOP_K
BM = 128
MAXB = TK // BM + E
PADP = MAXB * BM

NW = 32
TPW = T // NW
G = 4


def _routing_kernel(x_ref, gwt_ref, gam_ref, bet_ref,
                    xn_ref, pos_ref, w_ref, meta_ref, lb_ref, exc_ref):
    xv = x_ref[...]
    mu = jnp.mean(xv, axis=1, keepdims=True)
    var = jnp.mean((xv - mu) ** 2, axis=1, keepdims=True)
    xn = (xv - mu) / jnp.sqrt(var + LN_EPS) * gam_ref[...] + bet_ref[...]
    xn_ref[...] = xn

    scores = jnp.dot(xn.astype(jnp.bfloat16), gwt_ref[...].astype(jnp.bfloat16),
                     preferred_element_type=jnp.float32)
    m = jnp.max(scores, axis=1, keepdims=True)
    ex = jnp.exp(scores - m)
    p = ex / jnp.sum(ex, axis=1, keepdims=True)

    iota8 = lax.broadcasted_iota(jnp.int32, (T, E), 1)
    m0 = jnp.max(p, axis=1, keepdims=True)
    i0 = jnp.min(jnp.where(p == m0, iota8, E), axis=1, keepdims=True)
    oh0 = (iota8 == i0)
    p2 = jnp.where(oh0, -1.0, p)
    m1 = jnp.max(p2, axis=1, keepdims=True)
    i1 = jnp.min(jnp.where(p2 == m1, iota8, E), axis=1, keepdims=True)
    oh1 = (iota8 == i1)
    sw = m0 + m1
    w_ref[...] = jnp.concatenate([m0 / sw, m1 / sw], axis=1)

    oh0f = oh0.astype(jnp.float32)
    oh1f = oh1.astype(jnp.float32)
    s = oh0f + oh1f

    counts = jnp.sum(s, axis=0, keepdims=True)
    pmean = jnp.mean(p, axis=0, keepdims=True)
    lb_ref[...] = (LB_WEIGHT * E / TK) * jnp.sum(counts * pmean,
                                                 keepdims=True).reshape(1, 1)

    tri = (lax.broadcasted_iota(jnp.int32, (BM, BM), 0)
           > lax.broadcasted_iota(jnp.int32, (BM, BM), 1)).astype(jnp.float32)
    carry = jnp.zeros((1, E), jnp.float32)
    for c in range(T // BM):
        blk = s[c * BM:(c + 1) * BM, :]
        exc_ref[c * BM:(c + 1) * BM, :] = jnp.dot(
            tri, blk, preferred_element_type=jnp.float32,
            precision=lax.Precision.HIGHEST) + carry
        carry = carry + jnp.sum(blk, axis=0, keepdims=True)
    exc = exc_ref[...]

    rank0 = jnp.sum(oh0f * exc, axis=1, keepdims=True)
    rank1 = jnp.sum(oh1f * (exc + oh0f), axis=1, keepdims=True)

    ci = carry.astype(jnp.int32)
    pc = ((ci + (BM - 1)) >> 7) << 7
    pcf = pc.astype(jnp.float32)
    up8 = (lax.broadcasted_iota(jnp.int32, (E, E), 0)
           < lax.broadcasted_iota(jnp.int32, (E, E), 1)).astype(jnp.float32)
    po = jnp.dot(pcf, up8, preferred_element_type=jnp.float32,
                 precision=lax.Precision.HIGHEST)

    pos0 = jnp.sum(oh0f * po, axis=1, keepdims=True) + rank0
    pos1 = jnp.sum(oh1f * po, axis=1, keepdims=True) + rank1
    pos_ref[...] = jnp.concatenate([pos0, pos1], axis=1).astype(jnp.int32)

    po_b = jnp.broadcast_to(po * (1.0 / BM), (2 * E, E))
    pc_b = jnp.broadcast_to(pcf * (1.0 / BM), (2 * E, E))
    ir = lax.broadcasted_iota(jnp.int32, (2 * E, E), 0)
    ie = lax.broadcasted_iota(jnp.int32, (2 * E, E), 1)
    meta_ref[...] = jnp.sum(jnp.where(ie == ir, po_b, 0.0)
                            + jnp.where(ie == ir - E, pc_b, 0.0),
                            axis=1, keepdims=True).astype(jnp.int32)


def _sc_dispatch_kernel(xn_hbm, pos_hbm, out_hbm, pos_v, idx0_v, idx1_v,
                        rows_v, sem):
    wid = lax.axis_index("s") * 2 + lax.axis_index("c")
    base = wid * TPW
    pltpu.sync_copy(xn_hbm.at[pl.ds(base, TPW)], rows_v)
    pltpu.sync_copy(pos_hbm.at[pl.ds(2 * base, 2 * TPW)], pos_v)
    lanes = lax.iota(jnp.int32, 16)
    for c in range(TPW // 16):
        idx0_v[pl.ds(c * 16, 16)] = plsc.load_gather(
            pos_v, [c * 32 + lanes * 2])
        idx1_v[pl.ds(c * 16, 16)] = plsc.load_gather(
            pos_v, [c * 32 + lanes * 2 + 1])
    c0 = pltpu.async_copy(rows_v, out_hbm.at[idx0_v], sem)
    c1 = pltpu.async_copy(rows_v, out_hbm.at[idx1_v], sem)
    c0.wait()
    c1.wait()


def _ffn_kernel(m_ref, x_ref, uw_ref, ub_ref, dw_ref, db_ref,
                p0_ref, p1_ref, w0_ref, w1_ref, y_ref, acc_ref):
    e = pl.program_id(0)
    f = pl.program_id(1)
    pb_e = m_ref[e]
    nb_e = m_ref[E + e]
    uwc = uw_ref[0].astype(jnp.bfloat16)
    dwc = dw_ref[0].astype(jnp.bfloat16)
    ubc = ub_ref[0]
    dbc = db_ref[0]

    @pl.when((e == 0) & (f == 0))
    def _():
        y_ref[...] = jnp.zeros_like(y_ref)

    def blk_body(j, carry):
        rs = pl.ds((pb_e + j) * BM, BM)
        xb = x_ref[rs, :]
        xb = jnp.where(jnp.abs(xb) < 1e30, xb, 0.0)
        h = lax.dot_general(xb.astype(jnp.bfloat16), uwc,
                            (((1,), (1,)), ((), ())),
                            preferred_element_type=jnp.float32) + ubc
        g = 0.5 * h * (1.0 + lax.erf(h * 0.7071067811865476))
        part = lax.dot_general(g.astype(jnp.bfloat16), dwc,
                               (((1,), (1,)), ((), ())),
                               preferred_element_type=jnp.float32)

        @pl.when(f == 0)
        def _():
            acc_ref[rs, :] = part + dbc

        @pl.when((f != 0) & (f != G - 1))
        def _():
            acc_ref[rs, :] += part

        @pl.when(f == G - 1)
        def _():
            rows = (acc_ref[rs, :] + part).astype(jnp.bfloat16)
            col = ((pb_e + j) * BM
                   + lax.broadcasted_iota(jnp.int32, (1, BM), 1))
            cmat = (jnp.where(p0_ref[...] == col, w0_ref[...], 0.0)
                    + jnp.where(p1_ref[...] == col, w1_ref[...], 0.0))
            y_ref[...] += lax.dot_general(cmat.astype(jnp.bfloat16), rows,
                                          (((1,), (0,)), ((), ())),
                                          preferred_element_type=jnp.float32)

        return carry

    lax.fori_loop(0, nb_e, blk_body, 0)


def kernel(x, gate_W, ln_gamma, ln_beta, up_W, up_b, down_W, down_b):
    x2d = x.reshape(T, H)

    xn, pos2, wpair, meta_col, lb = pl.pallas_call(
        _routing_kernel,
        out_shape=(
            jax.ShapeDtypeStruct((T, H), jnp.float32),
            jax.ShapeDtypeStruct((T, TOP_K), jnp.int32),
            jax.ShapeDtypeStruct((T, TOP_K), jnp.float32),
            jax.ShapeDtypeStruct((2 * E, 1), jnp.int32),
            jax.ShapeDtypeStruct((1, 1), jnp.float32),
        ),
        scratch_shapes=[pltpu.VMEM((T, E), jnp.float32)],
    )(x2d, gate_W.T, ln_gamma.reshape(1, H), ln_beta.reshape(1, H))

    pos_flat = pos2.reshape(TK)
    meta = meta_col.reshape(2 * E)

    mesh = plsc.VectorSubcoreMesh(core_axis_name="c", subcore_axis_name="s")

    xs = pl.kernel(
        _sc_dispatch_kernel,
        out_type=jax.ShapeDtypeStruct((PADP, H), jnp.float32),
        mesh=mesh,
        scratch_types=[
            pltpu.VMEM((2 * TPW,), jnp.int32),
            pltpu.VMEM((TPW,), jnp.int32),
            pltpu.VMEM((TPW,), jnp.int32),
            pltpu.VMEM((TPW, H), jnp.float32),
            pltpu.SemaphoreType.DMA,
        ],
        compiler_params=pltpu.CompilerParams(needs_layout_passes=False),
    )(xn, pos_flat)

    y2d = pl.pallas_call(
        _ffn_kernel,
        grid_spec=pltpu.PrefetchScalarGridSpec(
            num_scalar_prefetch=1,
            grid=(E, G),
            in_specs=[
                pl.BlockSpec((PADP, H), lambda e, f, m: (0, 0)),
                pl.BlockSpec((1, HFF // G, H), lambda e, f, m: (e, f, 0)),
                pl.BlockSpec((1, 1, HFF // G), lambda e, f, m: (e, 0, f)),
                pl.BlockSpec((1, H, HFF // G), lambda e, f, m: (e, 0, f)),
                pl.BlockSpec((1, 1, H), lambda e, f, m: (e, 0, 0)),
                pl.BlockSpec((T, 1), lambda e, f, m: (0, 0)),
                pl.BlockSpec((T, 1), lambda e, f, m: (0, 0)),
                pl.BlockSpec((T, 1), lambda e, f, m: (0, 0)),
                pl.BlockSpec((T, 1), lambda e, f, m: (0, 0)),
            ],
            out_specs=pl.BlockSpec((T, H), lambda e, f, m: (0, 0)),
            scratch_shapes=[pltpu.VMEM((PADP, H), jnp.float32)],
        ),
        out_shape=jax.ShapeDtypeStruct((T, H), jnp.float32),
    )(meta, xs, up_W, up_b.reshape(E, 1, HFF), down_W, down_b.reshape(E, 1, H),
      pos2[:, 0:1], pos2[:, 1:2], wpair[:, 0:1], wpair[:, 1:2])

    return (y2d.reshape(1, T, H), lb.reshape(()))


# speedup vs baseline: 1.3333x; 1.3333x over previous
import jax
import jax.numpy as jnp
from jax import lax
from jax.experimental import pallas as pl
from jax.experimental.pallas import tpu as pltpu
from jax.experimental.pallas import tpu_sc as plsc

H = 768
E = 8
TOP_K = 2
HFF = 3072
LB_WEIGHT = 0.01
LN_EPS = 1e-5

T = 2048
TK = T * TOP_K
BM = 128
MAXB = TK // BM + E
PADP = MAXB * BM

NW = 32
TPW = T // NW


def _routing_kernel(x_ref, gwt_ref, gam_ref, bet_ref,
                    xn_ref, pos_ref, w_ref, meta_ref, lb_ref, exc_ref):
    xv = x_ref[...]
    mu = jnp.mean(xv, axis=1, keepdims=True)
    var = jnp.mean((xv - mu) ** 2, axis=1, keepdims=True)
    xn = (xv - mu) / jnp.sqrt(var + LN_EPS) * gam_ref[...] + bet_ref[...]
    xn_ref[...] = xn

    scores = jnp.dot(xn.astype(jnp.bfloat16), gwt_ref[...].astype(jnp.bfloat16),
                     preferred_element_type=jnp.float32)
    m = jnp.max(scores, axis=1, keepdims=True)
    ex = jnp.exp(scores - m)
    p = ex / jnp.sum(ex, axis=1, keepdims=True)

    iota8 = lax.broadcasted_iota(jnp.int32, (T, E), 1)
    m0 = jnp.max(p, axis=1, keepdims=True)
    i0 = jnp.min(jnp.where(p == m0, iota8, E), axis=1, keepdims=True)
    oh0 = (iota8 == i0)
    p2 = jnp.where(oh0, -1.0, p)
    m1 = jnp.max(p2, axis=1, keepdims=True)
    i1 = jnp.min(jnp.where(p2 == m1, iota8, E), axis=1, keepdims=True)
    oh1 = (iota8 == i1)
    sw = m0 + m1
    w_ref[...] = jnp.concatenate([m0 / sw, m1 / sw], axis=1)

    oh0f = oh0.astype(jnp.float32)
    oh1f = oh1.astype(jnp.float32)
    s = oh0f + oh1f

    counts = jnp.sum(s, axis=0, keepdims=True)
    pmean = jnp.mean(p, axis=0, keepdims=True)
    lb_ref[...] = (LB_WEIGHT * E / TK) * jnp.sum(counts * pmean,
                                                 keepdims=True).reshape(1, 1)

    tri = (lax.broadcasted_iota(jnp.int32, (BM, BM), 0)
           > lax.broadcasted_iota(jnp.int32, (BM, BM), 1)).astype(jnp.float32)
    carry = jnp.zeros((1, E), jnp.float32)
    for c in range(T // BM):
        blk = s[c * BM:(c + 1) * BM, :]
        exc_ref[c * BM:(c + 1) * BM, :] = jnp.dot(
            tri, blk, preferred_element_type=jnp.float32,
            precision=lax.Precision.HIGHEST) + carry
        carry = carry + jnp.sum(blk, axis=0, keepdims=True)
    exc = exc_ref[...]

    rank0 = jnp.sum(oh0f * exc, axis=1, keepdims=True)
    rank1 = jnp.sum(oh1f * (exc + oh0f), axis=1, keepdims=True)

    ci = carry.astype(jnp.int32)
    pc = ((ci + (BM - 1)) >> 7) << 7
    pcf = pc.astype(jnp.float32)
    up8 = (lax.broadcasted_iota(jnp.int32, (E, E), 0)
           < lax.broadcasted_iota(jnp.int32, (E, E), 1)).astype(jnp.float32)
    po = jnp.dot(pcf, up8, preferred_element_type=jnp.float32,
                 precision=lax.Precision.HIGHEST)

    pos0 = jnp.sum(oh0f * po, axis=1, keepdims=True) + rank0
    pos1 = jnp.sum(oh1f * po, axis=1, keepdims=True) + rank1
    pos_ref[...] = jnp.concatenate([pos0, pos1], axis=1).astype(jnp.int32)

    po_b = jnp.broadcast_to(po, (MAXB + 1, E))
    irow = (lax.broadcasted_iota(jnp.int32, (MAXB + 1, E), 0) * BM
            ).astype(jnp.float32)
    be = jnp.sum((po_b <= irow).astype(jnp.float32), axis=1,
                 keepdims=True) - 1.0
    nbtot = jnp.sum(pcf, axis=1, keepdims=True) * (1.0 / BM)
    is_last = lax.broadcasted_iota(jnp.int32, (MAXB + 1, 1), 0) == MAXB
    meta_ref[...] = jnp.where(is_last, jnp.broadcast_to(nbtot, (MAXB + 1, 1)),
                              be).astype(jnp.int32)


def _sc_dispatch_kernel(xn_hbm, pos_hbm, out_hbm, pos_v, idx0_v, idx1_v,
                        rows_v, sem):
    wid = lax.axis_index("s") * 2 + lax.axis_index("c")
    base = wid * TPW
    pltpu.sync_copy(xn_hbm.at[pl.ds(base, TPW)], rows_v)
    pltpu.sync_copy(pos_hbm.at[pl.ds(2 * base, 2 * TPW)], pos_v)
    lanes = lax.iota(jnp.int32, 16)
    for c in range(TPW // 16):
        idx0_v[pl.ds(c * 16, 16)] = plsc.load_gather(
            pos_v, [c * 32 + lanes * 2])
        idx1_v[pl.ds(c * 16, 16)] = plsc.load_gather(
            pos_v, [c * 32 + lanes * 2 + 1])
    c0 = pltpu.async_copy(rows_v, out_hbm.at[idx0_v], sem)
    c1 = pltpu.async_copy(rows_v, out_hbm.at[idx1_v], sem)
    c0.wait()
    c1.wait()


def _ffn_kernel(m_ref, x_ref, uw_ref, ub_ref, dw_ref, db_ref, o_ref,
                uwb_ref, dwb_ref):
    i = pl.program_id(0)
    nbt = m_ref[MAXB]
    ei = m_ref[jnp.minimum(i, nbt - 1)]
    prev = m_ref[jnp.minimum(jnp.maximum(i - 1, 0), nbt - 1)]

    @pl.when((i == 0) | (ei != prev))
    def _():
        uwb_ref[...] = uw_ref[0].astype(jnp.bfloat16)
        dwb_ref[...] = dw_ref[0].astype(jnp.bfloat16)

    @pl.when(i < nbt)
    def _():
        xb = x_ref[...]
        xb = jnp.where(jnp.abs(xb) < 1e30, xb, 0.0)
        h = lax.dot_general(xb.astype(jnp.bfloat16), uwb_ref[...],
                            (((1,), (1,)), ((), ())),
                            preferred_element_type=jnp.float32) + ub_ref[0]
        g = 0.5 * h * (1.0 + lax.erf(h * 0.7071067811865476))
        o_ref[...] = lax.dot_general(g.astype(jnp.bfloat16), dwb_ref[...],
                                     (((1,), (1,)), ((), ())),
                                     preferred_element_type=jnp.float32
                                     ) + db_ref[0]


def _sc_combine_kernel(os_hbm, pos_hbm, w_hbm, y_hbm, pos_v, idx0_v, idx1_v,
                       w_v, rowsA_v, rowsB_v, sem):
    wid = lax.axis_index("s") * 2 + lax.axis_index("c")
    base = wid * TPW
    pltpu.sync_copy(pos_hbm.at[pl.ds(2 * base, 2 * TPW)], pos_v)
    pltpu.sync_copy(w_hbm.at[pl.ds(2 * base, 2 * TPW)], w_v)
    lanes = lax.iota(jnp.int32, 16)
    for c in range(TPW // 16):
        idx0_v[pl.ds(c * 16, 16)] = plsc.load_gather(
            pos_v, [c * 32 + lanes * 2])
        idx1_v[pl.ds(c * 16, 16)] = plsc.load_gather(
            pos_v, [c * 32 + lanes * 2 + 1])
    g0 = pltpu.async_copy(os_hbm.at[idx0_v], rowsA_v, sem)
    g1 = pltpu.async_copy(os_hbm.at[idx1_v], rowsB_v, sem)
    g0.wait()
    g1.wait()

    z16 = jnp.zeros((16,), jnp.int32)

    def tok_body(t, carry):
        w0v = plsc.load_gather(w_v, [z16 + 2 * t])
        w1v = plsc.load_gather(w_v, [z16 + 2 * t + 1])
        for c in range(H // 16):
            sl = pl.ds(c * 16, 16)
            rowsA_v[t, sl] = (w0v * rowsA_v[t, sl] + w1v * rowsB_v[t, sl])
        return carry

    lax.fori_loop(0, TPW, tok_body, 0)
    pltpu.sync_copy(rowsA_v, y_hbm.at[pl.ds(base, TPW)])


def kernel(x, gate_W, ln_gamma, ln_beta, up_W, up_b, down_W, down_b):
    x2d = x.reshape(T, H)

    xn, pos2, wpair, meta_col, lb = pl.pallas_call(
        _routing_kernel,
        out_shape=(
            jax.ShapeDtypeStruct((T, H), jnp.float32),
            jax.ShapeDtypeStruct((T, TOP_K), jnp.int32),
            jax.ShapeDtypeStruct((T, TOP_K), jnp.float32),
            jax.ShapeDtypeStruct((MAXB + 1, 1), jnp.int32),
            jax.ShapeDtypeStruct((1, 1), jnp.float32),
        ),
        scratch_shapes=[pltpu.VMEM((T, E), jnp.float32)],
    )(x2d, gate_W.T, ln_gamma.reshape(1, H), ln_beta.reshape(1, H))

    pos_flat = pos2.reshape(TK)
    w_flat = wpair.reshape(TK)
    meta = meta_col.reshape(MAXB + 1)

    mesh = plsc.VectorSubcoreMesh(core_axis_name="c", subcore_axis_name="s")

    xs = pl.kernel(
        _sc_dispatch_kernel,
        out_type=jax.ShapeDtypeStruct((PADP, H), jnp.float32),
        mesh=mesh,
        scratch_types=[
            pltpu.VMEM((2 * TPW,), jnp.int32),
            pltpu.VMEM((TPW,), jnp.int32),
            pltpu.VMEM((TPW,), jnp.int32),
            pltpu.VMEM((TPW, H), jnp.float32),
            pltpu.SemaphoreType.DMA,
        ],
        compiler_params=pltpu.CompilerParams(needs_layout_passes=False),
    )(xn, pos_flat)

    os_ = pl.pallas_call(
        _ffn_kernel,
        grid_spec=pltpu.PrefetchScalarGridSpec(
            num_scalar_prefetch=1,
            grid=(MAXB,),
            in_specs=[
                pl.BlockSpec((BM, H), lambda i, m: (i, 0)),
                pl.BlockSpec((1, HFF, H),
                             lambda i, m: (m[jnp.minimum(i, m[MAXB] - 1)], 0, 0)),
                pl.BlockSpec((1, 1, HFF),
                             lambda i, m: (m[jnp.minimum(i, m[MAXB] - 1)], 0, 0)),
                pl.BlockSpec((1, H, HFF),
                             lambda i, m: (m[jnp.minimum(i, m[MAXB] - 1)], 0, 0)),
                pl.BlockSpec((1, 1, H),
                             lambda i, m: (m[jnp.minimum(i, m[MAXB] - 1)], 0, 0)),
            ],
            out_specs=pl.BlockSpec((BM, H), lambda i, m: (i, 0)),
            scratch_shapes=[pltpu.VMEM((HFF, H), jnp.bfloat16),
                            pltpu.VMEM((H, HFF), jnp.bfloat16)],
        ),
        out_shape=jax.ShapeDtypeStruct((PADP, H), jnp.float32),
    )(meta, xs, up_W, up_b.reshape(E, 1, HFF), down_W, down_b.reshape(E, 1, H))

    y2d = pl.kernel(
        _sc_combine_kernel,
        out_type=jax.ShapeDtypeStruct((T, H), jnp.float32),
        mesh=mesh,
        scratch_types=[
            pltpu.VMEM((2 * TPW,), jnp.int32),
            pltpu.VMEM((TPW,), jnp.int32),
            pltpu.VMEM((TPW,), jnp.int32),
            pltpu.VMEM((2 * TPW,), jnp.float32),
            pltpu.VMEM((TPW, H), jnp.float32),
            pltpu.VMEM((TPW, H), jnp.float32),
            pltpu.SemaphoreType.DMA,
        ],
        compiler_params=pltpu.CompilerParams(needs_layout_passes=False),
    )(os_, pos_flat, w_flat)

    return (y2d.reshape(1, T, H), lb.reshape(()))
